# baseline (speedup 1.0000x reference)
import functools

import jax
import jax.numpy as jnp
from jax import lax
from jax.experimental import pallas as pl
from jax.experimental.pallas import tpu as pltpu
from jax.experimental.pallas import tpu_sc as plsc

NC = 2
NS = 16
NW = NC * NS
LANES = 16
CHUNK = 128

N_U = 5000
N_I = 5000
D = 128
E = 320000

NP = 5008
RPT = NP // NS
HN = 5120
PADIDX = 5000
CPT = 2 * (-(-E // (NW * CHUNK * 2)))
SLAB = CPT + 1
EPAD = NW * CPT * CHUNK


_mesh = plsc.VectorSubcoreMesh(
    core_axis_name="c", subcore_axis_name="s", num_cores=NC, num_subcores=NS
)


def _hist_body(src_hbm, dst_hbm, hist_hbm, idx_v, sub_v, deg_v):
    c = lax.axis_index("c")
    s = lax.axis_index("s")
    wid = c * NS + s
    lane = lax.broadcasted_iota(jnp.int32, (LANES,), 0)
    ones = jnp.ones((LANES,), jnp.float32)
    zeros = jnp.zeros((LANES,), jnp.float32)

    for d, ref in ((0, src_hbm), (1, dst_hbm)):
        pltpu.sync_copy(ref.at[wid], idx_v)

        def zero_body(t, _):
            r = t // (HN // LANES)
            k = t % (HN // LANES)
            sub_v[r, pl.ds(k * LANES, LANES)] = zeros
            return _

        lax.fori_loop(0, NS * (HN // LANES), zero_body, 0)

        def edge_body(t, _):
            j = t // (CHUNK // LANES)
            k = t % (CHUNK // LANES)
            idx = idx_v[j, pl.ds(k * LANES, LANES)]
            plsc.addupdate_scatter(sub_v, [lane, idx], ones)
            return _

        lax.fori_loop(0, CPT * (CHUNK // LANES), edge_body, 0)

        def red_body(i, _):
            acc = sub_v[0, pl.ds(i * LANES, LANES)]
            for r in range(1, NS):
                acc = acc + sub_v[r, pl.ds(i * LANES, LANES)]
            deg_v[d, pl.ds(i * LANES, LANES)] = acc
            return _

        lax.fori_loop(0, HN // LANES, red_body, 0)

    pltpu.sync_copy(deg_v.at[0], hist_hbm.at[wid])
    pltpu.sync_copy(deg_v.at[1], hist_hbm.at[NW + wid])


_hist_call = pl.kernel(
    _hist_body,
    out_type=jax.ShapeDtypeStruct((2 * NW, HN), jnp.float32),
    mesh=_mesh,
    scratch_types=[
        pltpu.VMEM((SLAB, CHUNK), jnp.int32),
        pltpu.VMEM((NS, HN), jnp.float32),
        pltpu.VMEM((2, HN), jnp.float32),
    ],
    compiler_params=pltpu.CompilerParams(use_tc_tiling_on_sc=False, needs_layout_passes=False),
)


def _prep_body(hist_ref, u_ref, i_ref, wu_ref, wi_ref, inv_ref):
    h = hist_ref[...]
    r = lax.broadcasted_iota(jnp.int32, (2 * NW, 2), 0)
    col = lax.broadcasted_iota(jnp.int32, (2 * NW, 2), 1)
    sel = jnp.where((r < NW) == (col == 0), 1.0, 0.0).astype(jnp.float32)
    deg2 = lax.dot_general(
        h, sel, (((0,), (0,)), ((), ())), preferred_element_type=jnp.float32
    )
    inv2 = jnp.where(deg2 > 0, lax.rsqrt(jnp.maximum(deg2, 1.0)), 0.0)
    inv_ref[...] = inv2
    wu_ref[...] = u_ref[...] * inv2[:NP, 0:1]
    wi_ref[...] = i_ref[...] * inv2[:NP, 1:2]


_prep_call = pl.pallas_call(
    _prep_body,
    out_shape=[
        jax.ShapeDtypeStruct((NP, D), jnp.float32),
        jax.ShapeDtypeStruct((NP, D), jnp.float32),
        jax.ShapeDtypeStruct((HN, 2), jnp.float32),
    ],
)


def _main_body(
    wu_hbm, wi_hbm, src_hbm, dst_hbm, oi_hbm, ou_hbm,
    srcv, dstv, bufu, bufi, acc, sem_u, sem_i,
):
    c = lax.axis_index("c")
    s = lax.axis_index("s")
    wid = c * NS + s
    pltpu.sync_copy(src_hbm.at[wid], srcv)
    pltpu.sync_copy(dst_hbm.at[wid], dstv)

    zeros = jnp.zeros((LANES,), jnp.float32)

    def zero_body(t, _):
        r = t // (D // LANES)
        k = t % (D // LANES)
        bufu[r, pl.ds(k * LANES, LANES)] = zeros
        return _

    lax.fori_loop(0, CHUNK * (D // LANES), zero_body, 0)

    row0 = s * RPT
    tail = RPT - 2 * CHUNK

    def zero_acc():
        pltpu.sync_copy(bufu, acc.at[pl.ds(row0, CHUNK)])
        pltpu.sync_copy(bufu, acc.at[pl.ds(row0 + CHUNK, CHUNK)])
        pltpu.sync_copy(bufu.at[pl.ds(0, tail)], acc.at[pl.ds(row0 + 2 * CHUNK, tail)])

    off = c * NP + row0

    def run_pass(table_hbm, gidx, sidx):
        def body(j, _):
            pltpu.async_copy(table_hbm.at[gidx.at[j]], bufu, sem_u).wait()
            pltpu.sync_copy(bufu, acc.at[sidx.at[j]], add=True)
            return _

        lax.fori_loop(0, CPT, body, 0)

    zero_acc()
    plsc.subcore_barrier()
    run_pass(wu_hbm, srcv, dstv)
    plsc.subcore_barrier()
    pltpu.sync_copy(acc.at[pl.ds(row0, RPT)], oi_hbm.at[pl.ds(off, RPT)])

    def rezero_body(t, _):
        r = t // (D // LANES)
        k = t % (D // LANES)
        bufu[r, pl.ds(k * LANES, LANES)] = zeros
        return _

    lax.fori_loop(0, CHUNK * (D // LANES), rezero_body, 0)
    zero_acc()
    plsc.subcore_barrier()
    run_pass(wi_hbm, dstv, srcv)
    plsc.subcore_barrier()
    pltpu.sync_copy(acc.at[pl.ds(row0, RPT)], ou_hbm.at[pl.ds(off, RPT)])


_main_call = pl.kernel(
    _main_body,
    out_type=[
        jax.ShapeDtypeStruct((NC * NP, D), jnp.float32),
        jax.ShapeDtypeStruct((NC * NP, D), jnp.float32),
    ],
    mesh=_mesh,
    scratch_types=[
        pltpu.VMEM((SLAB, CHUNK), jnp.int32),
        pltpu.VMEM((SLAB, CHUNK), jnp.int32),
        pltpu.VMEM((CHUNK, D), jnp.float32),
        pltpu.VMEM((CHUNK, D), jnp.float32),
        pltpu.VMEM_SHARED((NP, D), jnp.float32),
        pltpu.SemaphoreType.DMA,
        pltpu.SemaphoreType.DMA,
    ],
    compiler_params=pltpu.CompilerParams(use_tc_tiling_on_sc=False, needs_layout_passes=False),
)


def _fin_body(oi_ref, ou_ref, inv_ref, items_ref, users_ref):
    inv2 = inv_ref[...]
    items_ref[...] = (oi_ref[0:NP, :] + oi_ref[NP : 2 * NP, :]) * inv2[:NP, 1:2]
    users_ref[...] = (ou_ref[0:NP, :] + ou_ref[NP : 2 * NP, :]) * inv2[:NP, 0:1]


_fin_call = pl.pallas_call(
    _fin_body,
    out_shape=[
        jax.ShapeDtypeStruct((NP, D), jnp.float32),
        jax.ShapeDtypeStruct((NP, D), jnp.float32),
    ],
)


@jax.jit
def kernel(ufeats, ifeats, edge_index):
    src = edge_index[0].astype(jnp.int32)
    dst = edge_index[1].astype(jnp.int32)
    pad = jnp.full((EPAD - E,), PADIDX, jnp.int32)
    dummy = jnp.full((NW, SLAB - CPT, CHUNK), PADIDX, jnp.int32)
    src2 = jnp.concatenate(
        [jnp.concatenate([src, pad]).reshape(NW, CPT, CHUNK), dummy], axis=1)
    dst2 = jnp.concatenate(
        [jnp.concatenate([dst, pad]).reshape(NW, CPT, CHUNK), dummy], axis=1)
    zrows = jnp.zeros((NP - N_U, D), jnp.float32)
    up = jnp.concatenate([ufeats, zrows], axis=0)
    ip = jnp.concatenate([ifeats, zrows], axis=0)

    hist = _hist_call(src2, dst2)
    wu, wi, inv2 = _prep_call(hist, up, ip)
    oi, ou = _main_call(wu, wi, src2, dst2)
    items, users = _fin_call(oi, ou, inv2)
    return users[:N_U], items[:N_I]


# speedup vs baseline: 1.4958x; 1.4958x over previous
import functools

import jax
import jax.numpy as jnp
from jax import lax
from jax.experimental import pallas as pl
from jax.experimental.pallas import tpu as pltpu
from jax.experimental.pallas import tpu_sc as plsc

NC = 2
NS = 16
NW = NC * NS
LANES = 16
CHUNK = 128

N_U = 5000
N_I = 5000
D = 128
E = 320000

NP = 5008
RPT = NP // NS
HN = 5120
PADIDX = 5000
CPT = -(-E // (NW * CHUNK))
SLAB = CPT
EPAD = NW * CPT * CHUNK


_mesh = plsc.VectorSubcoreMesh(
    core_axis_name="c", subcore_axis_name="s", num_cores=NC, num_subcores=NS
)


def _hist_body(src_hbm, dst_hbm, hist_hbm, idx_v, sub_v, deg_v):
    c = lax.axis_index("c")
    s = lax.axis_index("s")
    wid = c * NS + s
    lane = lax.broadcasted_iota(jnp.int32, (LANES,), 0)
    ones = jnp.ones((LANES,), jnp.float32)
    zeros = jnp.zeros((LANES,), jnp.float32)

    for d, ref in ((0, src_hbm), (1, dst_hbm)):
        pltpu.sync_copy(ref.at[wid], idx_v)

        def zero_body(t, _):
            r = t // (HN // LANES)
            k = t % (HN // LANES)
            sub_v[r, pl.ds(k * LANES, LANES)] = zeros
            return _

        lax.fori_loop(0, NS * (HN // LANES), zero_body, 0)

        def edge_body(t, _):
            j = t // (CHUNK // LANES)
            k = t % (CHUNK // LANES)
            idx = idx_v[j, pl.ds(k * LANES, LANES)]
            plsc.addupdate_scatter(sub_v, [lane, idx], ones)
            return _

        lax.fori_loop(0, CPT * (CHUNK // LANES), edge_body, 0)

        def red_body(i, _):
            acc = sub_v[0, pl.ds(i * LANES, LANES)]
            for r in range(1, NS):
                acc = acc + sub_v[r, pl.ds(i * LANES, LANES)]
            deg_v[d, pl.ds(i * LANES, LANES)] = acc
            return _

        lax.fori_loop(0, HN // LANES, red_body, 0)

    pltpu.sync_copy(deg_v.at[0], hist_hbm.at[wid])
    pltpu.sync_copy(deg_v.at[1], hist_hbm.at[NW + wid])


_hist_call = pl.kernel(
    _hist_body,
    out_type=jax.ShapeDtypeStruct((2 * NW, HN), jnp.float32),
    mesh=_mesh,
    scratch_types=[
        pltpu.VMEM((SLAB, CHUNK), jnp.int32),
        pltpu.VMEM((NS, HN), jnp.float32),
        pltpu.VMEM((2, HN), jnp.float32),
    ],
    compiler_params=pltpu.CompilerParams(use_tc_tiling_on_sc=False, needs_layout_passes=False),
)


def _prep_body(hist_ref, u_ref, i_ref, wu_ref, wi_ref, inv_ref):
    h = hist_ref[...]
    r = lax.broadcasted_iota(jnp.int32, (2 * NW, 2), 0)
    col = lax.broadcasted_iota(jnp.int32, (2 * NW, 2), 1)
    sel = jnp.where((r < NW) == (col == 0), 1.0, 0.0).astype(jnp.float32)
    deg2 = lax.dot_general(
        h, sel, (((0,), (0,)), ((), ())), preferred_element_type=jnp.float32
    )
    inv2 = jnp.where(deg2 > 0, lax.rsqrt(jnp.maximum(deg2, 1.0)), 0.0)
    inv_ref[...] = inv2
    wu_ref[...] = u_ref[...] * inv2[:NP, 0:1]
    wi_ref[...] = i_ref[...] * inv2[:NP, 1:2]


_prep_call = pl.pallas_call(
    _prep_body,
    out_shape=[
        jax.ShapeDtypeStruct((NP, D), jnp.float32),
        jax.ShapeDtypeStruct((NP, D), jnp.float32),
        jax.ShapeDtypeStruct((HN, 2), jnp.float32),
    ],
)


def _main_body(
    wu_hbm, wi_hbm, src_hbm, dst_hbm, oi_hbm, ou_hbm,
    srcv, dstv, bufu, bufi, acc, sem_u, sem_i,
):
    c = lax.axis_index("c")
    s = lax.axis_index("s")
    wid = c * NS + s
    pltpu.sync_copy(src_hbm.at[wid], srcv)
    pltpu.sync_copy(dst_hbm.at[wid], dstv)

    zeros = jnp.zeros((LANES,), jnp.float32)

    def zero_body(t, _):
        r = t // (D // LANES)
        k = t % (D // LANES)
        bufu[r, pl.ds(k * LANES, LANES)] = zeros
        return _

    lax.fori_loop(0, CHUNK * (D // LANES), zero_body, 0)

    row0 = s * RPT
    tail = RPT - 2 * CHUNK

    def zero_acc():
        pltpu.sync_copy(bufu, acc.at[pl.ds(row0, CHUNK)])
        pltpu.sync_copy(bufu, acc.at[pl.ds(row0 + CHUNK, CHUNK)])
        pltpu.sync_copy(bufu.at[pl.ds(0, tail)], acc.at[pl.ds(row0 + 2 * CHUNK, tail)])

    off = c * NP + row0

    def run_pass(table_hbm, gidx, sidx):
        def body(j, _):
            pltpu.async_copy(table_hbm.at[gidx.at[j]], bufu, sem_u).wait()
            pltpu.sync_copy(bufu, acc.at[sidx.at[j]], add=True)
            return _

        lax.fori_loop(0, CPT, body, 0)

    zero_acc()
    plsc.subcore_barrier()
    run_pass(wu_hbm, srcv, dstv)
    plsc.subcore_barrier()
    pltpu.sync_copy(acc.at[pl.ds(row0, RPT)], oi_hbm.at[pl.ds(off, RPT)])

    def rezero_body(t, _):
        r = t // (D // LANES)
        k = t % (D // LANES)
        bufu[r, pl.ds(k * LANES, LANES)] = zeros
        return _

    lax.fori_loop(0, CHUNK * (D // LANES), rezero_body, 0)
    zero_acc()
    plsc.subcore_barrier()
    run_pass(wi_hbm, dstv, srcv)
    plsc.subcore_barrier()
    pltpu.sync_copy(acc.at[pl.ds(row0, RPT)], ou_hbm.at[pl.ds(off, RPT)])


_main_call = pl.kernel(
    _main_body,
    out_type=[
        jax.ShapeDtypeStruct((NC * NP, D), jnp.float32),
        jax.ShapeDtypeStruct((NC * NP, D), jnp.float32),
    ],
    mesh=_mesh,
    scratch_types=[
        pltpu.VMEM((SLAB, CHUNK), jnp.int32),
        pltpu.VMEM((SLAB, CHUNK), jnp.int32),
        pltpu.VMEM((CHUNK, D), jnp.float32),
        pltpu.VMEM((CHUNK, D), jnp.float32),
        pltpu.VMEM_SHARED((NP, D), jnp.float32),
        pltpu.SemaphoreType.DMA,
        pltpu.SemaphoreType.DMA,
    ],
    compiler_params=pltpu.CompilerParams(use_tc_tiling_on_sc=False, needs_layout_passes=False),
)


def _fin_body(oi_ref, ou_ref, inv_ref, items_ref, users_ref):
    inv2 = inv_ref[...]
    items_ref[...] = (oi_ref[0:NP, :] + oi_ref[NP : 2 * NP, :]) * inv2[:NP, 1:2]
    users_ref[...] = (ou_ref[0:NP, :] + ou_ref[NP : 2 * NP, :]) * inv2[:NP, 0:1]


_fin_call = pl.pallas_call(
    _fin_body,
    out_shape=[
        jax.ShapeDtypeStruct((NP, D), jnp.float32),
        jax.ShapeDtypeStruct((NP, D), jnp.float32),
    ],
)


@jax.jit
def kernel(ufeats, ifeats, edge_index):
    src = edge_index[0].astype(jnp.int32)
    dst = edge_index[1].astype(jnp.int32)
    pad = jnp.full((EPAD - E,), PADIDX, jnp.int32)
    src2 = jnp.concatenate([src, pad]).reshape(NW, SLAB, CHUNK)
    dst2 = jnp.concatenate([dst, pad]).reshape(NW, SLAB, CHUNK)
    zrows = jnp.zeros((NP - N_U, D), jnp.float32)
    up = jnp.concatenate([ufeats, zrows], axis=0)
    ip = jnp.concatenate([ifeats, zrows], axis=0)

    hist = _hist_call(src2, dst2)
    wu, wi, inv2 = _prep_call(hist, up, ip)
    oi, ou = _main_call(wu, wi, src2, dst2)
    items, users = _fin_call(oi, ou, inv2)
    return users[:N_U], items[:N_I]


# speedup vs baseline: 2.5643x; 1.7144x over previous
import functools

import numpy as _np

import jax
import jax.numpy as jnp
from jax import lax
from jax.experimental import pallas as pl
from jax.experimental.pallas import tpu as pltpu
from jax.experimental.pallas import tpu_sc as plsc

NC = 2
NS = 16
NW = NC * NS
LANES = 16
CHUNK = 128

N_U = 5000
N_I = 5000
D = 128
E = 320000

NP = 5008
RPT = NP // NS
HN = 5120
NCHUNK = E // CHUNK
BASE, REM = divmod(NCHUNK, NW)
SLAB = BASE + 1

_perm = _np.full((NW, SLAB), NCHUNK, _np.int32)
for _w in range(NW):
    _off = _w * BASE + min(_w, REM)
    _cnt = BASE + (1 if _w < REM else 0)
    _perm[_w, :_cnt] = _np.arange(_off, _off + _cnt)


_mesh = plsc.VectorSubcoreMesh(
    core_axis_name="c", subcore_axis_name="s", num_cores=NC, num_subcores=NS
)


def _hist_body(src_hbm, dst_hbm, hist_hbm, idx_v, sub_v, deg_v):
    c = lax.axis_index("c")
    s = lax.axis_index("s")
    wid = c * NS + s
    cnt = BASE + jnp.where(wid < REM, 1, 0)
    lane = lax.broadcasted_iota(jnp.int32, (LANES,), 0)
    ones = jnp.ones((LANES,), jnp.float32)
    zeros = jnp.zeros((LANES,), jnp.float32)

    for d, ref in ((0, src_hbm), (1, dst_hbm)):
        pltpu.sync_copy(ref.at[wid], idx_v)

        def zero_body(t, _):
            r = t // (HN // LANES)
            k = t % (HN // LANES)
            sub_v[r, pl.ds(k * LANES, LANES)] = zeros
            return _

        lax.fori_loop(0, NS * (HN // LANES), zero_body, 0)

        def edge_body(t, _):
            j = t // (CHUNK // LANES)
            k = t % (CHUNK // LANES)
            idx = idx_v[j, pl.ds(k * LANES, LANES)]
            plsc.addupdate_scatter(sub_v, [lane, idx], ones)
            return _

        lax.fori_loop(0, cnt * (CHUNK // LANES), edge_body, 0)

        def red_body(i, _):
            acc = sub_v[0, pl.ds(i * LANES, LANES)]
            for r in range(1, NS):
                acc = acc + sub_v[r, pl.ds(i * LANES, LANES)]
            deg_v[d, pl.ds(i * LANES, LANES)] = acc
            return _

        lax.fori_loop(0, HN // LANES, red_body, 0)

    pltpu.sync_copy(deg_v.at[0], hist_hbm.at[wid])
    pltpu.sync_copy(deg_v.at[1], hist_hbm.at[NW + wid])


_hist_call = pl.kernel(
    _hist_body,
    out_type=jax.ShapeDtypeStruct((2 * NW, HN), jnp.float32),
    mesh=_mesh,
    scratch_types=[
        pltpu.VMEM((SLAB, CHUNK), jnp.int32),
        pltpu.VMEM((NS, HN), jnp.float32),
        pltpu.VMEM((2, HN), jnp.float32),
    ],
    compiler_params=pltpu.CompilerParams(use_tc_tiling_on_sc=False, needs_layout_passes=False),
)


def _prep_body(hist_ref, u_ref, i_ref, wu_ref, wi_ref, inv_ref):
    h = hist_ref[...]
    r = lax.broadcasted_iota(jnp.int32, (2 * NW, 2), 0)
    col = lax.broadcasted_iota(jnp.int32, (2 * NW, 2), 1)
    sel = jnp.where((r < NW) == (col == 0), 1.0, 0.0).astype(jnp.float32)
    deg2 = lax.dot_general(
        h, sel, (((0,), (0,)), ((), ())), preferred_element_type=jnp.float32
    )
    inv2 = jnp.where(deg2 > 0, lax.rsqrt(jnp.maximum(deg2, 1.0)), 0.0)
    inv_ref[...] = inv2
    wu_ref[...] = u_ref[...] * inv2[:NP, 0:1]
    wi_ref[...] = i_ref[...] * inv2[:NP, 1:2]


_prep_call = pl.pallas_call(
    _prep_body,
    out_shape=[
        jax.ShapeDtypeStruct((NP, D), jnp.float32),
        jax.ShapeDtypeStruct((NP, D), jnp.float32),
        jax.ShapeDtypeStruct((HN, 2), jnp.float32),
    ],
)


def _main_body(
    wu_hbm, wi_hbm, src_hbm, dst_hbm, oi_hbm, ou_hbm,
    srcv, dstv, bufu, bufi, acc, sem_u, sem_i,
):
    c = lax.axis_index("c")
    s = lax.axis_index("s")
    wid = c * NS + s
    cnt = BASE + jnp.where(wid < REM, 1, 0)
    pltpu.sync_copy(src_hbm.at[wid], srcv)
    pltpu.sync_copy(dst_hbm.at[wid], dstv)

    zeros = jnp.zeros((LANES,), jnp.float32)

    def zero_body(t, _):
        r = t // (D // LANES)
        k = t % (D // LANES)
        bufu[r, pl.ds(k * LANES, LANES)] = zeros
        return _

    lax.fori_loop(0, CHUNK * (D // LANES), zero_body, 0)

    row0 = s * RPT
    tail = RPT - 2 * CHUNK

    def zero_acc():
        pltpu.sync_copy(bufu, acc.at[pl.ds(row0, CHUNK)])
        pltpu.sync_copy(bufu, acc.at[pl.ds(row0 + CHUNK, CHUNK)])
        pltpu.sync_copy(bufu.at[pl.ds(0, tail)], acc.at[pl.ds(row0 + 2 * CHUNK, tail)])

    off = c * NP + row0

    def run_pass(table_hbm, gidx, sidx):
        def body(j, _):
            pltpu.async_copy(table_hbm.at[gidx.at[j]], bufu, sem_u).wait()
            pltpu.sync_copy(bufu, acc.at[sidx.at[j]], add=True)
            return _

        lax.fori_loop(0, cnt, body, 0)

    zero_acc()
    plsc.subcore_barrier()
    run_pass(wu_hbm, srcv, dstv)
    plsc.subcore_barrier()
    pltpu.sync_copy(acc.at[pl.ds(row0, RPT)], oi_hbm.at[pl.ds(off, RPT)])

    def rezero_body(t, _):
        r = t // (D // LANES)
        k = t % (D // LANES)
        bufu[r, pl.ds(k * LANES, LANES)] = zeros
        return _

    lax.fori_loop(0, CHUNK * (D // LANES), rezero_body, 0)
    zero_acc()
    plsc.subcore_barrier()
    run_pass(wi_hbm, dstv, srcv)
    plsc.subcore_barrier()
    pltpu.sync_copy(acc.at[pl.ds(row0, RPT)], ou_hbm.at[pl.ds(off, RPT)])


_main_call = pl.kernel(
    _main_body,
    out_type=[
        jax.ShapeDtypeStruct((NC * NP, D), jnp.float32),
        jax.ShapeDtypeStruct((NC * NP, D), jnp.float32),
    ],
    mesh=_mesh,
    scratch_types=[
        pltpu.VMEM((SLAB, CHUNK), jnp.int32),
        pltpu.VMEM((SLAB, CHUNK), jnp.int32),
        pltpu.VMEM((CHUNK, D), jnp.float32),
        pltpu.VMEM((CHUNK, D), jnp.float32),
        pltpu.VMEM_SHARED((NP, D), jnp.float32),
        pltpu.SemaphoreType.DMA,
        pltpu.SemaphoreType.DMA,
    ],
    compiler_params=pltpu.CompilerParams(use_tc_tiling_on_sc=False, needs_layout_passes=False),
)


def _fin_body(oi_ref, ou_ref, inv_ref, items_ref, users_ref):
    inv2 = inv_ref[...]
    items_ref[...] = (oi_ref[0:NP, :] + oi_ref[NP : 2 * NP, :]) * inv2[:NP, 1:2]
    users_ref[...] = (ou_ref[0:NP, :] + ou_ref[NP : 2 * NP, :]) * inv2[:NP, 0:1]


_fin_call = pl.pallas_call(
    _fin_body,
    out_shape=[
        jax.ShapeDtypeStruct((NP, D), jnp.float32),
        jax.ShapeDtypeStruct((NP, D), jnp.float32),
    ],
)


@jax.jit
def kernel(ufeats, ifeats, edge_index):
    src = edge_index[0].astype(jnp.int32)
    dst = edge_index[1].astype(jnp.int32)
    perm = jnp.asarray(_perm)
    dummy = jnp.zeros((1, CHUNK), jnp.int32)
    src2 = jnp.concatenate([src.reshape(NCHUNK, CHUNK), dummy])[perm]
    dst2 = jnp.concatenate([dst.reshape(NCHUNK, CHUNK), dummy])[perm]
    zrows = jnp.zeros((NP - N_U, D), jnp.float32)
    up = jnp.concatenate([ufeats, zrows], axis=0)
    ip = jnp.concatenate([ifeats, zrows], axis=0)

    hist = _hist_call(src2, dst2)
    wu, wi, inv2 = _prep_call(hist, up, ip)
    oi, ou = _main_call(wu, wi, src2, dst2)
    items, users = _fin_call(oi, ou, inv2)
    return users[:N_U], items[:N_I]


# speedup vs baseline: 2.8880x; 1.1262x over previous
import functools

import numpy as _np

import jax
import jax.numpy as jnp
from jax import lax
from jax.experimental import pallas as pl
from jax.experimental.pallas import tpu as pltpu
from jax.experimental.pallas import tpu_sc as plsc

NC = 2
NS = 16
NW = NC * NS
LANES = 16
CHUNK = 128

N_U = 5000
N_I = 5000
D = 128
E = 320000

NP = 5008
RPT = NP // NS
HN = 5120
NCHUNK = E // CHUNK
BASE, REM = divmod(NCHUNK, NW)
SLAB = BASE + 1

_perm = _np.full((NW, SLAB), NCHUNK, _np.int32)
for _w in range(NW):
    _off = _w * BASE + min(_w, REM)
    _cnt = BASE + (1 if _w < REM else 0)
    _perm[_w, :_cnt] = _np.arange(_off, _off + _cnt)


_mesh = plsc.VectorSubcoreMesh(
    core_axis_name="c", subcore_axis_name="s", num_cores=NC, num_subcores=NS
)


def _hist_body(src_hbm, dst_hbm, hist_hbm, idx_v, sub_v, deg_v):
    c = lax.axis_index("c")
    s = lax.axis_index("s")
    wid = c * NS + s
    cnt = BASE + jnp.where(wid < REM, 1, 0)
    lane = lax.broadcasted_iota(jnp.int32, (LANES,), 0)
    ones = jnp.ones((LANES,), jnp.float32)
    zeros = jnp.zeros((LANES,), jnp.float32)

    for d, ref in ((0, src_hbm), (1, dst_hbm)):
        pltpu.sync_copy(ref.at[wid], idx_v)

        def zero_body(t, _):
            r = t // (HN // LANES)
            k = t % (HN // LANES)
            sub_v[r, pl.ds(k * LANES, LANES)] = zeros
            return _

        lax.fori_loop(0, NS * (HN // LANES), zero_body, 0)

        def edge_body(t, _):
            j = t // (CHUNK // LANES)
            k = t % (CHUNK // LANES)
            idx = idx_v[j, pl.ds(k * LANES, LANES)]
            plsc.addupdate_scatter(sub_v, [lane, idx], ones)
            return _

        lax.fori_loop(0, cnt * (CHUNK // LANES), edge_body, 0)

        def red_body(i, _):
            acc = sub_v[0, pl.ds(i * LANES, LANES)]
            for r in range(1, NS):
                acc = acc + sub_v[r, pl.ds(i * LANES, LANES)]
            deg_v[d, pl.ds(i * LANES, LANES)] = acc
            return _

        lax.fori_loop(0, HN // LANES, red_body, 0)

    pltpu.sync_copy(deg_v.at[0], hist_hbm.at[wid])
    pltpu.sync_copy(deg_v.at[1], hist_hbm.at[NW + wid])


_hist_call = pl.kernel(
    _hist_body,
    out_type=jax.ShapeDtypeStruct((2 * NW, HN), jnp.float32),
    mesh=_mesh,
    scratch_types=[
        pltpu.VMEM((SLAB, CHUNK), jnp.int32),
        pltpu.VMEM((NS, HN), jnp.float32),
        pltpu.VMEM((2, HN), jnp.float32),
    ],
    compiler_params=pltpu.CompilerParams(use_tc_tiling_on_sc=False, needs_layout_passes=False),
)


def _prep_body(hist_ref, u_ref, i_ref, wu_ref, wi_ref, inv_ref):
    h = hist_ref[...]
    r = lax.broadcasted_iota(jnp.int32, (2 * NW, 2), 0)
    col = lax.broadcasted_iota(jnp.int32, (2 * NW, 2), 1)
    sel = jnp.where((r < NW) == (col == 0), 1.0, 0.0).astype(jnp.float32)
    deg2 = lax.dot_general(
        h, sel, (((0,), (0,)), ((), ())), preferred_element_type=jnp.float32
    )
    inv2 = jnp.where(deg2 > 0, lax.rsqrt(jnp.maximum(deg2, 1.0)), 0.0)
    inv_ref[...] = inv2
    wu_ref[...] = u_ref[...] * inv2[:NP, 0:1]
    wi_ref[...] = i_ref[...] * inv2[:NP, 1:2]


_prep_call = pl.pallas_call(
    _prep_body,
    out_shape=[
        jax.ShapeDtypeStruct((NP, D), jnp.float32),
        jax.ShapeDtypeStruct((NP, D), jnp.float32),
        jax.ShapeDtypeStruct((HN, 2), jnp.float32),
    ],
)


def _main_body(
    wu_hbm, wi_hbm, src_hbm, dst_hbm, oi_hbm, ou_hbm,
    srcv, dstv, bufu, bufi, acc, sem_u, sem_i,
):
    c = lax.axis_index("c")
    s = lax.axis_index("s")
    wid = c * NS + s
    cnt = BASE + jnp.where(wid < REM, 1, 0)
    pltpu.sync_copy(src_hbm.at[wid], srcv)
    pltpu.sync_copy(dst_hbm.at[wid], dstv)

    zeros = jnp.zeros((LANES,), jnp.float32)

    def zero_body(t, _):
        r = t // (D // LANES)
        k = t % (D // LANES)
        bufu[r, pl.ds(k * LANES, LANES)] = zeros
        return _

    lax.fori_loop(0, CHUNK * (D // LANES), zero_body, 0)

    row0 = s * RPT
    tail = RPT - 2 * CHUNK

    def zero_acc():
        pltpu.sync_copy(bufu, acc.at[pl.ds(row0, CHUNK)])
        pltpu.sync_copy(bufu, acc.at[pl.ds(row0 + CHUNK, CHUNK)])
        pltpu.sync_copy(bufu.at[pl.ds(0, tail)], acc.at[pl.ds(row0 + 2 * CHUNK, tail)])

    off = c * NP + row0

    def run_pass(table_hbm, gidx, sidx):
        def body(j2, _):
            j = 2 * j2
            da = pltpu.async_copy(table_hbm.at[gidx.at[j]], bufu, sem_u)
            db = pltpu.async_copy(table_hbm.at[gidx.at[j + 1]], bufi, sem_i)
            da.wait()
            pltpu.sync_copy(bufu, acc.at[sidx.at[j]], add=True)
            db.wait()
            pltpu.sync_copy(bufi, acc.at[sidx.at[j + 1]], add=True)
            return _

        lax.fori_loop(0, cnt // 2, body, 0)

        @pl.when(cnt % 2 == 1)
        def _tail():
            pltpu.async_copy(table_hbm.at[gidx.at[cnt - 1]], bufu, sem_u).wait()
            pltpu.sync_copy(bufu, acc.at[sidx.at[cnt - 1]], add=True)

    zero_acc()
    plsc.subcore_barrier()
    run_pass(wu_hbm, srcv, dstv)
    plsc.subcore_barrier()
    pltpu.sync_copy(acc.at[pl.ds(row0, RPT)], oi_hbm.at[pl.ds(off, RPT)])

    def rezero_body(t, _):
        r = t // (D // LANES)
        k = t % (D // LANES)
        bufu[r, pl.ds(k * LANES, LANES)] = zeros
        return _

    lax.fori_loop(0, CHUNK * (D // LANES), rezero_body, 0)
    zero_acc()
    plsc.subcore_barrier()
    run_pass(wi_hbm, dstv, srcv)
    plsc.subcore_barrier()
    pltpu.sync_copy(acc.at[pl.ds(row0, RPT)], ou_hbm.at[pl.ds(off, RPT)])


_main_call = pl.kernel(
    _main_body,
    out_type=[
        jax.ShapeDtypeStruct((NC * NP, D), jnp.float32),
        jax.ShapeDtypeStruct((NC * NP, D), jnp.float32),
    ],
    mesh=_mesh,
    scratch_types=[
        pltpu.VMEM((SLAB, CHUNK), jnp.int32),
        pltpu.VMEM((SLAB, CHUNK), jnp.int32),
        pltpu.VMEM((CHUNK, D), jnp.float32),
        pltpu.VMEM((CHUNK, D), jnp.float32),
        pltpu.VMEM_SHARED((NP, D), jnp.float32),
        pltpu.SemaphoreType.DMA,
        pltpu.SemaphoreType.DMA,
    ],
    compiler_params=pltpu.CompilerParams(use_tc_tiling_on_sc=False, needs_layout_passes=False),
)


def _fin_body(oi_ref, ou_ref, inv_ref, items_ref, users_ref):
    inv2 = inv_ref[...]
    items_ref[...] = (oi_ref[0:NP, :] + oi_ref[NP : 2 * NP, :]) * inv2[:NP, 1:2]
    users_ref[...] = (ou_ref[0:NP, :] + ou_ref[NP : 2 * NP, :]) * inv2[:NP, 0:1]


_fin_call = pl.pallas_call(
    _fin_body,
    out_shape=[
        jax.ShapeDtypeStruct((NP, D), jnp.float32),
        jax.ShapeDtypeStruct((NP, D), jnp.float32),
    ],
)


@jax.jit
def kernel(ufeats, ifeats, edge_index):
    src = edge_index[0].astype(jnp.int32)
    dst = edge_index[1].astype(jnp.int32)
    perm = jnp.asarray(_perm)
    dummy = jnp.zeros((1, CHUNK), jnp.int32)
    src2 = jnp.concatenate([src.reshape(NCHUNK, CHUNK), dummy])[perm]
    dst2 = jnp.concatenate([dst.reshape(NCHUNK, CHUNK), dummy])[perm]
    zrows = jnp.zeros((NP - N_U, D), jnp.float32)
    up = jnp.concatenate([ufeats, zrows], axis=0)
    ip = jnp.concatenate([ifeats, zrows], axis=0)

    hist = _hist_call(src2, dst2)
    wu, wi, inv2 = _prep_call(hist, up, ip)
    oi, ou = _main_call(wu, wi, src2, dst2)
    items, users = _fin_call(oi, ou, inv2)
    return users[:N_U], items[:N_I]


# speedup vs baseline: 3.0072x; 1.0413x over previous
import numpy as _np

import jax
import jax.numpy as jnp
from jax import lax
from jax.experimental import pallas as pl
from jax.experimental.pallas import tpu as pltpu
from jax.experimental.pallas import tpu_sc as plsc

NC = 2
NS = 16
NW = NC * NS
LANES = 16

N_U = 5000
N_I = 5000
D = 128
E = 320000

NP = 5008
RPT = NP // NS
HN = 5120
EPT = E // NW
CHUNK = 125
CPT = EPT // CHUNK
VPT = EPT // LANES


_mesh = plsc.VectorSubcoreMesh(
    core_axis_name="c", subcore_axis_name="s", num_cores=NC, num_subcores=NS
)

_sc_params = pltpu.CompilerParams(
    use_tc_tiling_on_sc=False, needs_layout_passes=False
)


def _hist_body(src_hbm, dst_hbm, hist_hbm, idx_v, sub_v, deg_v):
    c = lax.axis_index("c")
    s = lax.axis_index("s")
    wid = c * NS + s
    lane = lax.broadcasted_iota(jnp.int32, (LANES,), 0)
    ones = jnp.ones((LANES,), jnp.float32)
    zeros = jnp.zeros((LANES,), jnp.float32)

    for d, ref in ((0, src_hbm), (1, dst_hbm)):
        pltpu.sync_copy(ref.at[wid], idx_v)

        def zero_body(t, _):
            sub_v[pl.ds(t * LANES, LANES)] = zeros
            return _

        lax.fori_loop(0, NS * (HN // LANES), zero_body, 0)

        def edge_body(t, _):
            idx = idx_v[pl.ds(t * LANES, LANES)]
            plsc.addupdate_scatter(sub_v, [lane * HN + idx], ones)
            return _

        lax.fori_loop(0, VPT, edge_body, 0)

        def red_body(i, _):
            acc = sub_v[pl.ds(i * LANES, LANES)]
            for r in range(1, NS):
                acc = acc + sub_v[pl.ds(r * HN + i * LANES, LANES)]
            deg_v[pl.ds(d * HN + i * LANES, LANES)] = acc
            return _

        lax.fori_loop(0, HN // LANES, red_body, 0)

    pltpu.sync_copy(deg_v.at[pl.ds(0, HN)], hist_hbm.at[wid])
    pltpu.sync_copy(deg_v.at[pl.ds(HN, HN)], hist_hbm.at[NW + wid])


_hist_call = pl.kernel(
    _hist_body,
    out_type=jax.ShapeDtypeStruct((2 * NW, HN), jnp.float32),
    mesh=_mesh,
    scratch_types=[
        pltpu.VMEM((EPT,), jnp.int32),
        pltpu.VMEM((NS * HN,), jnp.float32),
        pltpu.VMEM((2 * HN,), jnp.float32),
    ],
    compiler_params=_sc_params,
)


def _prep_body(hist_ref, u_ref, i_ref, wu_ref, wi_ref, inv_ref):
    h = hist_ref[...]
    r = lax.broadcasted_iota(jnp.int32, (2 * NW, 2), 0)
    col = lax.broadcasted_iota(jnp.int32, (2 * NW, 2), 1)
    sel = jnp.where((r < NW) == (col == 0), 1.0, 0.0).astype(jnp.float32)
    deg2 = lax.dot_general(
        h, sel, (((0,), (0,)), ((), ())), preferred_element_type=jnp.float32
    )
    inv2 = jnp.where(deg2 > 0, lax.rsqrt(jnp.maximum(deg2, 1.0)), 0.0)
    inv_ref[...] = inv2
    wu_ref[...] = u_ref[...] * inv2[:NP, 0:1]
    wi_ref[...] = i_ref[...] * inv2[:NP, 1:2]


_prep_call = pl.pallas_call(
    _prep_body,
    out_shape=[
        jax.ShapeDtypeStruct((NP, D), jnp.float32),
        jax.ShapeDtypeStruct((NP, D), jnp.float32),
        jax.ShapeDtypeStruct((HN, 2), jnp.float32),
    ],
)


def _main_body(
    wu_hbm, wi_hbm, src_hbm, dst_hbm, oi_hbm, ou_hbm,
    srcv, dstv, bufu, bufi, acc, sem_u, sem_i,
):
    c = lax.axis_index("c")
    s = lax.axis_index("s")
    wid = c * NS + s
    pltpu.sync_copy(src_hbm.at[wid], srcv)
    pltpu.sync_copy(dst_hbm.at[wid], dstv)

    zeros = jnp.zeros((LANES,), jnp.float32)

    def zero_buf():
        def zero_body(t, _):
            r = t // (D // LANES)
            k = t % (D // LANES)
            bufu[r, pl.ds(k * LANES, LANES)] = zeros
            return _

        lax.fori_loop(0, CHUNK * (D // LANES), zero_body, 0)

    row0 = s * RPT
    tail = RPT - 2 * CHUNK

    def zero_acc():
        pltpu.sync_copy(bufu, acc.at[pl.ds(row0, CHUNK)])
        pltpu.sync_copy(bufu, acc.at[pl.ds(row0 + CHUNK, CHUNK)])
        pltpu.sync_copy(bufu.at[pl.ds(0, tail)], acc.at[pl.ds(row0 + 2 * CHUNK, tail)])

    off = c * NP + row0

    def run_pass(table_hbm, gidx, sidx):
        def body(j2, _):
            j = 2 * j2
            da = pltpu.async_copy(table_hbm.at[gidx.at[j]], bufu, sem_u)
            db = pltpu.async_copy(table_hbm.at[gidx.at[j + 1]], bufi, sem_i)
            da.wait()
            pltpu.sync_copy(bufu, acc.at[sidx.at[j]], add=True)
            db.wait()
            pltpu.sync_copy(bufi, acc.at[sidx.at[j + 1]], add=True)
            return _

        lax.fori_loop(0, CPT // 2, body, 0)

    zero_buf()
    zero_acc()
    plsc.subcore_barrier()
    run_pass(wu_hbm, srcv, dstv)
    plsc.subcore_barrier()
    pltpu.sync_copy(acc.at[pl.ds(row0, RPT)], oi_hbm.at[pl.ds(off, RPT)])

    zero_buf()
    zero_acc()
    plsc.subcore_barrier()
    run_pass(wi_hbm, dstv, srcv)
    plsc.subcore_barrier()
    pltpu.sync_copy(acc.at[pl.ds(row0, RPT)], ou_hbm.at[pl.ds(off, RPT)])


_main_call = pl.kernel(
    _main_body,
    out_type=[
        jax.ShapeDtypeStruct((NC * NP, D), jnp.float32),
        jax.ShapeDtypeStruct((NC * NP, D), jnp.float32),
    ],
    mesh=_mesh,
    scratch_types=[
        pltpu.VMEM((CPT, CHUNK), jnp.int32),
        pltpu.VMEM((CPT, CHUNK), jnp.int32),
        pltpu.VMEM((CHUNK, D), jnp.float32),
        pltpu.VMEM((CHUNK, D), jnp.float32),
        pltpu.VMEM_SHARED((NP, D), jnp.float32),
        pltpu.SemaphoreType.DMA,
        pltpu.SemaphoreType.DMA,
    ],
    compiler_params=_sc_params,
)


def _fin_body(oi_ref, ou_ref, inv_ref, items_ref, users_ref):
    inv2 = inv_ref[...]
    items_ref[...] = (oi_ref[0:NP, :] + oi_ref[NP : 2 * NP, :]) * inv2[:NP, 1:2]
    users_ref[...] = (ou_ref[0:NP, :] + ou_ref[NP : 2 * NP, :]) * inv2[:NP, 0:1]


_fin_call = pl.pallas_call(
    _fin_body,
    out_shape=[
        jax.ShapeDtypeStruct((NP, D), jnp.float32),
        jax.ShapeDtypeStruct((NP, D), jnp.float32),
    ],
)


@jax.jit
def kernel(ufeats, ifeats, edge_index):
    src = edge_index[0].astype(jnp.int32)
    dst = edge_index[1].astype(jnp.int32)
    src3 = src.reshape(NW, CPT, CHUNK)
    dst3 = dst.reshape(NW, CPT, CHUNK)
    srcf = src.reshape(NW, EPT)
    dstf = dst.reshape(NW, EPT)
    zrows = jnp.zeros((NP - N_U, D), jnp.float32)
    up = jnp.concatenate([ufeats, zrows], axis=0)
    ip = jnp.concatenate([ifeats, zrows], axis=0)

    hist = _hist_call(srcf, dstf)
    wu, wi, inv2 = _prep_call(hist, up, ip)
    oi, ou = _main_call(wu, wi, src3, dst3)
    items, users = _fin_call(oi, ou, inv2)
    return users[:N_U], items[:N_I]


# speedup vs baseline: 3.4026x; 1.1315x over previous
import numpy as _np

import jax
import jax.numpy as jnp
from jax import lax
from jax.experimental import pallas as pl
from jax.experimental.pallas import tpu as pltpu
from jax.experimental.pallas import tpu_sc as plsc

NC = 2
NS = 16
NW = NC * NS
LANES = 16

N_U = 5000
N_I = 5000
D = 128
E = 320000

NP = 5008
RPT = NP // NS
HN = 5120
EPT = E // NW
CHUNK = 125
CPT = EPT // CHUNK
VPT = EPT // LANES


_mesh = plsc.VectorSubcoreMesh(
    core_axis_name="c", subcore_axis_name="s", num_cores=NC, num_subcores=NS
)

_sc_params = pltpu.CompilerParams(
    use_tc_tiling_on_sc=False, needs_layout_passes=False
)


def _hist_body(src_hbm, dst_hbm, hist_hbm, idx_v, sub_v, deg_v):
    c = lax.axis_index("c")
    s = lax.axis_index("s")
    wid = c * NS + s
    lane = lax.broadcasted_iota(jnp.int32, (LANES,), 0)
    ones = jnp.ones((LANES,), jnp.float32)
    zeros = jnp.zeros((LANES,), jnp.float32)

    for d, ref in ((0, src_hbm), (1, dst_hbm)):
        pltpu.sync_copy(ref.at[wid], idx_v)

        def zero_body(t, _):
            for u in range(8):
                sub_v[pl.ds((t * 8 + u) * LANES, LANES)] = zeros
            return _

        lax.fori_loop(0, NS * (HN // LANES) // 8, zero_body, 0)

        def edge_body(t, _):
            for u in range(5):
                idx = idx_v[pl.ds((t * 5 + u) * LANES, LANES)]
                plsc.addupdate_scatter(sub_v, [lane * HN + idx], ones)
            return _

        lax.fori_loop(0, VPT // 5, edge_body, 0)

        def red_body(i, _):
            acc = sub_v[pl.ds(i * LANES, LANES)]
            for r in range(1, NS):
                acc = acc + sub_v[pl.ds(r * HN + i * LANES, LANES)]
            deg_v[pl.ds(d * HN + i * LANES, LANES)] = acc
            return _

        lax.fori_loop(0, HN // LANES, red_body, 0)

    pltpu.sync_copy(deg_v.at[pl.ds(0, HN)], hist_hbm.at[wid])
    pltpu.sync_copy(deg_v.at[pl.ds(HN, HN)], hist_hbm.at[NW + wid])


_hist_call = pl.kernel(
    _hist_body,
    out_type=jax.ShapeDtypeStruct((2 * NW, HN), jnp.float32),
    mesh=_mesh,
    scratch_types=[
        pltpu.VMEM((EPT,), jnp.int32),
        pltpu.VMEM((NS * HN,), jnp.float32),
        pltpu.VMEM((2 * HN,), jnp.float32),
    ],
    compiler_params=_sc_params,
)


def _prep_body(hist_ref, u_ref, i_ref, wu_ref, wi_ref, inv_ref):
    h = hist_ref[...]
    r = lax.broadcasted_iota(jnp.int32, (2 * NW, 2), 0)
    col = lax.broadcasted_iota(jnp.int32, (2 * NW, 2), 1)
    sel = jnp.where((r < NW) == (col == 0), 1.0, 0.0).astype(jnp.float32)
    deg2 = lax.dot_general(
        h, sel, (((0,), (0,)), ((), ())), preferred_element_type=jnp.float32
    )
    inv2 = jnp.where(deg2 > 0, lax.rsqrt(jnp.maximum(deg2, 1.0)), 0.0)
    inv_ref[...] = inv2
    wu_ref[...] = u_ref[...] * inv2[:NP, 0:1]
    wi_ref[...] = i_ref[...] * inv2[:NP, 1:2]


_prep_call = pl.pallas_call(
    _prep_body,
    out_shape=[
        jax.ShapeDtypeStruct((NP, D), jnp.float32),
        jax.ShapeDtypeStruct((NP, D), jnp.float32),
        jax.ShapeDtypeStruct((HN, 2), jnp.float32),
    ],
)


def _main_body(
    wu_hbm, wi_hbm, src_hbm, dst_hbm, oi_hbm, ou_hbm,
    srcv, dstv, bufu, bufi, acc, sem_u, sem_i,
):
    c = lax.axis_index("c")
    s = lax.axis_index("s")
    wid = c * NS + s
    pltpu.sync_copy(src_hbm.at[wid], srcv)
    pltpu.sync_copy(dst_hbm.at[wid], dstv)

    zeros = jnp.zeros((LANES,), jnp.float32)

    def zero_buf():
        def zero_body(r, _):
            for k in range(D // LANES):
                bufu[r, pl.ds(k * LANES, LANES)] = zeros
            return _

        lax.fori_loop(0, CHUNK, zero_body, 0)

    row0 = s * RPT
    tail = RPT - 2 * CHUNK

    def zero_acc():
        pltpu.sync_copy(bufu, acc.at[pl.ds(row0, CHUNK)])
        pltpu.sync_copy(bufu, acc.at[pl.ds(row0 + CHUNK, CHUNK)])
        pltpu.sync_copy(bufu.at[pl.ds(0, tail)], acc.at[pl.ds(row0 + 2 * CHUNK, tail)])

    off = c * NP + row0

    def run_pass(table_hbm, gidx, sidx):
        def body(j2, _):
            j = 2 * j2
            da = pltpu.async_copy(table_hbm.at[gidx.at[j]], bufu, sem_u)
            db = pltpu.async_copy(table_hbm.at[gidx.at[j + 1]], bufi, sem_i)
            da.wait()
            pltpu.sync_copy(bufu, acc.at[sidx.at[j]], add=True)
            db.wait()
            pltpu.sync_copy(bufi, acc.at[sidx.at[j + 1]], add=True)
            return _

        lax.fori_loop(0, CPT // 2, body, 0)

    zero_buf()
    zero_acc()
    plsc.subcore_barrier()
    run_pass(wu_hbm, srcv, dstv)
    plsc.subcore_barrier()
    pltpu.sync_copy(acc.at[pl.ds(row0, RPT)], oi_hbm.at[pl.ds(off, RPT)])

    zero_buf()
    zero_acc()
    plsc.subcore_barrier()
    run_pass(wi_hbm, dstv, srcv)
    plsc.subcore_barrier()
    pltpu.sync_copy(acc.at[pl.ds(row0, RPT)], ou_hbm.at[pl.ds(off, RPT)])


_main_call = pl.kernel(
    _main_body,
    out_type=[
        jax.ShapeDtypeStruct((NC * NP, D), jnp.float32),
        jax.ShapeDtypeStruct((NC * NP, D), jnp.float32),
    ],
    mesh=_mesh,
    scratch_types=[
        pltpu.VMEM((CPT, CHUNK), jnp.int32),
        pltpu.VMEM((CPT, CHUNK), jnp.int32),
        pltpu.VMEM((CHUNK, D), jnp.float32),
        pltpu.VMEM((CHUNK, D), jnp.float32),
        pltpu.VMEM_SHARED((NP, D), jnp.float32),
        pltpu.SemaphoreType.DMA,
        pltpu.SemaphoreType.DMA,
    ],
    compiler_params=_sc_params,
)


def _fin_body(oi_ref, ou_ref, inv_ref, items_ref, users_ref):
    inv2 = inv_ref[...]
    items_ref[...] = (oi_ref[0:NP, :] + oi_ref[NP : 2 * NP, :]) * inv2[:NP, 1:2]
    users_ref[...] = (ou_ref[0:NP, :] + ou_ref[NP : 2 * NP, :]) * inv2[:NP, 0:1]


_fin_call = pl.pallas_call(
    _fin_body,
    out_shape=[
        jax.ShapeDtypeStruct((NP, D), jnp.float32),
        jax.ShapeDtypeStruct((NP, D), jnp.float32),
    ],
)


@jax.jit
def kernel(ufeats, ifeats, edge_index):
    src = edge_index[0].astype(jnp.int32)
    dst = edge_index[1].astype(jnp.int32)
    src3 = src.reshape(NW, CPT, CHUNK)
    dst3 = dst.reshape(NW, CPT, CHUNK)
    srcf = src.reshape(NW, EPT)
    dstf = dst.reshape(NW, EPT)
    zrows = jnp.zeros((NP - N_U, D), jnp.float32)
    up = jnp.concatenate([ufeats, zrows], axis=0)
    ip = jnp.concatenate([ifeats, zrows], axis=0)

    hist = _hist_call(srcf, dstf)
    wu, wi, inv2 = _prep_call(hist, up, ip)
    oi, ou = _main_call(wu, wi, src3, dst3)
    items, users = _fin_call(oi, ou, inv2)
    return users[:N_U], items[:N_I]


# speedup vs baseline: 3.5340x; 1.0386x over previous
import numpy as _np

import jax
import jax.numpy as jnp
from jax import lax
from jax.experimental import pallas as pl
from jax.experimental.pallas import tpu as pltpu
from jax.experimental.pallas import tpu_sc as plsc

NC = 2
NS = 16
NW = NC * NS
LANES = 16

N_U = 5000
N_I = 5000
D = 128
E = 320000

NP = 5008
RPT = NP // NS
HN = 5120
EPT = E // NW
CHUNK = 125
CPT = EPT // CHUNK
VPT = EPT // LANES


_mesh = plsc.VectorSubcoreMesh(
    core_axis_name="c", subcore_axis_name="s", num_cores=NC, num_subcores=NS
)

_sc_params = pltpu.CompilerParams(
    use_tc_tiling_on_sc=False, needs_layout_passes=False
)


def _hist_body(src_hbm, dst_hbm, hist_hbm, idx_v, sub_v, deg_v):
    c = lax.axis_index("c")
    s = lax.axis_index("s")
    wid = c * NS + s
    lane = lax.broadcasted_iota(jnp.int32, (LANES,), 0)
    ones = jnp.ones((LANES,), jnp.float32)
    zeros = jnp.zeros((LANES,), jnp.float32)

    for d, ref in ((0, src_hbm), (1, dst_hbm)):
        pltpu.sync_copy(ref.at[wid], idx_v)

        def zero_body(t, _):
            for u in range(8):
                sub_v[pl.ds((t * 8 + u) * LANES, LANES)] = zeros
            return _

        lax.fori_loop(0, NS * (HN // LANES) // 8, zero_body, 0)

        def edge_body(t, _):
            for u in range(5):
                idx = idx_v[pl.ds((t * 5 + u) * LANES, LANES)]
                plsc.addupdate_scatter(sub_v, [lane * HN + idx], ones)
            return _

        lax.fori_loop(0, VPT // 5, edge_body, 0)

        def red_body(i, _):
            acc = sub_v[pl.ds(i * LANES, LANES)]
            for r in range(1, NS):
                acc = acc + sub_v[pl.ds(r * HN + i * LANES, LANES)]
            deg_v[pl.ds(d * HN + i * LANES, LANES)] = acc
            return _

        lax.fori_loop(0, HN // LANES, red_body, 0)

    pltpu.sync_copy(deg_v.at[pl.ds(0, HN)], hist_hbm.at[wid])
    pltpu.sync_copy(deg_v.at[pl.ds(HN, HN)], hist_hbm.at[NW + wid])


_hist_call = pl.kernel(
    _hist_body,
    out_type=jax.ShapeDtypeStruct((2 * NW, HN), jnp.float32),
    mesh=_mesh,
    scratch_types=[
        pltpu.VMEM((EPT,), jnp.int32),
        pltpu.VMEM((NS * HN,), jnp.float32),
        pltpu.VMEM((2 * HN,), jnp.float32),
    ],
    compiler_params=_sc_params,
)


def _prep_body(hist_ref, u_ref, i_ref, wu_ref, wi_ref, inv_ref):
    h = hist_ref[...]
    r = lax.broadcasted_iota(jnp.int32, (2 * NW, 2), 0)
    col = lax.broadcasted_iota(jnp.int32, (2 * NW, 2), 1)
    sel = jnp.where((r < NW) == (col == 0), 1.0, 0.0).astype(jnp.float32)
    deg2 = lax.dot_general(
        h, sel, (((0,), (0,)), ((), ())), preferred_element_type=jnp.float32
    )
    inv2 = jnp.where(deg2 > 0, lax.rsqrt(jnp.maximum(deg2, 1.0)), 0.0)
    inv_ref[...] = inv2
    wu_ref[...] = u_ref[...] * inv2[:NP, 0:1]
    wi_ref[...] = i_ref[...] * inv2[:NP, 1:2]


_prep_call = pl.pallas_call(
    _prep_body,
    out_shape=[
        jax.ShapeDtypeStruct((NP, D), jnp.float32),
        jax.ShapeDtypeStruct((NP, D), jnp.float32),
        jax.ShapeDtypeStruct((HN, 2), jnp.float32),
    ],
)


def _main_body(
    wu_hbm, wi_hbm, src_hbm, dst_hbm, oi_hbm, ou_hbm,
    srcv, dstv, bufu, bufi, bufc, bufd, acc, sem_u, sem_i, sem_c, sem_d,
):
    c = lax.axis_index("c")
    s = lax.axis_index("s")
    wid = c * NS + s
    pltpu.sync_copy(src_hbm.at[wid], srcv)
    pltpu.sync_copy(dst_hbm.at[wid], dstv)

    zeros = jnp.zeros((LANES,), jnp.float32)

    def zero_buf():
        def zero_body(r, _):
            for k in range(D // LANES):
                bufu[r, pl.ds(k * LANES, LANES)] = zeros
            return _

        lax.fori_loop(0, CHUNK, zero_body, 0)

    row0 = s * RPT
    tail = RPT - 2 * CHUNK

    def zero_acc():
        pltpu.sync_copy(bufu, acc.at[pl.ds(row0, CHUNK)])
        pltpu.sync_copy(bufu, acc.at[pl.ds(row0 + CHUNK, CHUNK)])
        pltpu.sync_copy(bufu.at[pl.ds(0, tail)], acc.at[pl.ds(row0 + 2 * CHUNK, tail)])

    off = c * NP + row0

    bufs = (bufu, bufi, bufc, bufd)
    sems = (sem_u, sem_i, sem_c, sem_d)

    def run_pass(table_hbm, gidx, sidx):
        def body(j4, _):
            j = 4 * j4
            ds_ = [
                pltpu.async_copy(table_hbm.at[gidx.at[j + u]], bufs[u], sems[u])
                for u in range(4)
            ]
            for u in range(4):
                ds_[u].wait()
                pltpu.sync_copy(bufs[u], acc.at[sidx.at[j + u]], add=True)
            return _

        lax.fori_loop(0, CPT // 4, body, 0)

    zero_buf()
    zero_acc()
    plsc.subcore_barrier()
    run_pass(wu_hbm, srcv, dstv)
    plsc.subcore_barrier()
    pltpu.sync_copy(acc.at[pl.ds(row0, RPT)], oi_hbm.at[pl.ds(off, RPT)])

    zero_buf()
    zero_acc()
    plsc.subcore_barrier()
    run_pass(wi_hbm, dstv, srcv)
    plsc.subcore_barrier()
    pltpu.sync_copy(acc.at[pl.ds(row0, RPT)], ou_hbm.at[pl.ds(off, RPT)])


_main_call = pl.kernel(
    _main_body,
    out_type=[
        jax.ShapeDtypeStruct((NC * NP, D), jnp.float32),
        jax.ShapeDtypeStruct((NC * NP, D), jnp.float32),
    ],
    mesh=_mesh,
    scratch_types=[
        pltpu.VMEM((CPT, CHUNK), jnp.int32),
        pltpu.VMEM((CPT, CHUNK), jnp.int32),
        pltpu.VMEM((CHUNK, D), jnp.float32),
        pltpu.VMEM((CHUNK, D), jnp.float32),
        pltpu.VMEM((CHUNK, D), jnp.float32),
        pltpu.VMEM((CHUNK, D), jnp.float32),
        pltpu.VMEM_SHARED((NP, D), jnp.float32),
        pltpu.SemaphoreType.DMA,
        pltpu.SemaphoreType.DMA,
        pltpu.SemaphoreType.DMA,
        pltpu.SemaphoreType.DMA,
    ],
    compiler_params=_sc_params,
)


def _fin_body(oi_ref, ou_ref, inv_ref, items_ref, users_ref):
    inv2 = inv_ref[...]
    items_ref[...] = (oi_ref[0:NP, :] + oi_ref[NP : 2 * NP, :]) * inv2[:NP, 1:2]
    users_ref[...] = (ou_ref[0:NP, :] + ou_ref[NP : 2 * NP, :]) * inv2[:NP, 0:1]


_fin_call = pl.pallas_call(
    _fin_body,
    out_shape=[
        jax.ShapeDtypeStruct((NP, D), jnp.float32),
        jax.ShapeDtypeStruct((NP, D), jnp.float32),
    ],
)


@jax.jit
def kernel(ufeats, ifeats, edge_index):
    src = edge_index[0].astype(jnp.int32)
    dst = edge_index[1].astype(jnp.int32)
    src3 = src.reshape(NW, CPT, CHUNK)
    dst3 = dst.reshape(NW, CPT, CHUNK)
    srcf = src.reshape(NW, EPT)
    dstf = dst.reshape(NW, EPT)
    zrows = jnp.zeros((NP - N_U, D), jnp.float32)
    up = jnp.concatenate([ufeats, zrows], axis=0)
    ip = jnp.concatenate([ifeats, zrows], axis=0)

    hist = _hist_call(srcf, dstf)
    wu, wi, inv2 = _prep_call(hist, up, ip)
    oi, ou = _main_call(wu, wi, src3, dst3)
    items, users = _fin_call(oi, ou, inv2)
    return users[:N_U], items[:N_I]
